# Initial kernel scaffold; baseline (speedup 1.0000x reference)
import jax
import jax.numpy as jnp
from jax.experimental import pallas as pl


def kernel(rx, r_edge_attr, r_edge_index, r_batch, r_g, px, p_edge_attr, p_edge_index, p_batch, params):
    raise NotImplementedError("write your pallas kernel here")


# speedup vs baseline: 2.1224x; 2.1224x over previous
import functools

import jax
import jax.numpy as jnp
from jax import lax
from jax.experimental import pallas as pl
from jax.experimental.pallas import tpu as pltpu
from jax.experimental.pallas import tpu_sc as plsc

N = 10000
E = 160000
G = 16
NF = 128

NCORES = 2
NSUB = 16
NW = NCORES * NSUB

G_RPW = 2 * E // NW
G_CH = 80
G_NK = G_RPW // G_CH

S_RPW = E // NW
S_CH = 40
S_NK = S_RPW // S_CH

BE = 2000
NB_E = E // BE
BN = 1000
NB_N = N // BN

_f32 = jnp.float32


def _relu(v):
    return jnp.maximum(v, 0.0)


def _sc_mesh():
    return plsc.VectorSubcoreMesh(core_axis_name="c", subcore_axis_name="s")


def _sc_gather(table, idx3):
    def body(table_ref, idx_ref, out_ref, idxv, rowsv, sem):
        cid = lax.axis_index("c")
        sid = lax.axis_index("s")
        wid = sid * NCORES + cid
        pltpu.sync_copy(idx_ref.at[wid], idxv)

        @pl.loop(0, G_NK)
        def _step(k):
            pltpu.async_copy(table_ref.at[idxv.at[k]], rowsv, sem).wait()
            pltpu.sync_copy(rowsv, out_ref.at[pl.ds(wid * G_RPW + k * G_CH, G_CH)])

    return pl.kernel(
        body,
        out_type=jax.ShapeDtypeStruct((2 * E, 128), _f32),
        mesh=_sc_mesh(),
        scratch_types=[
            pltpu.VMEM((G_NK, G_CH), jnp.int32),
            pltpu.VMEM((G_CH, 128), _f32),
            pltpu.SemaphoreType.DMA,
        ],
        name="sc_gather_x",
    )(table, idx3)


def _sc_scatter(m, col3, zeros):
    def body(m_ref, col_ref, z_ref, out_ref, idxv, rowsv, acc):
        cid = lax.axis_index("c")
        sid = lax.axis_index("s")
        wid = sid * NCORES + cid

        @pl.when(sid == 0)
        def _zero():
            pltpu.sync_copy(z_ref, acc)

        plsc.subcore_barrier()
        pltpu.sync_copy(col_ref.at[wid], idxv)

        @pl.loop(0, S_NK)
        def _step(k):
            pltpu.sync_copy(m_ref.at[pl.ds(wid * S_RPW + k * S_CH, S_CH)], rowsv)
            pltpu.sync_copy(rowsv, acc.at[idxv.at[k]], add=True)

        plsc.subcore_barrier()

        @pl.when(sid == 0)
        def _flush():
            pltpu.sync_copy(acc, out_ref.at[cid])

    return pl.kernel(
        body,
        out_type=jax.ShapeDtypeStruct((2, N, 128), _f32),
        mesh=_sc_mesh(),
        scratch_types=[
            pltpu.VMEM((S_NK, S_CH), jnp.int32),
            pltpu.VMEM((S_CH, 128), _f32),
            pltpu.VMEM_SHARED((N, 128), _f32),
        ],
        name="sc_scatter_m",
    )(m, col3, zeros)


def _sc_counts(col3, zeros, ones128):
    def body(col_ref, z_ref, o_ref, out_ref, idxv, onesv, acc):
        cid = lax.axis_index("c")
        sid = lax.axis_index("s")
        wid = sid * NCORES + cid

        @pl.when(sid == 0)
        def _zero():
            pltpu.sync_copy(z_ref, acc)

        plsc.subcore_barrier()
        pltpu.sync_copy(col_ref.at[wid], idxv)
        pltpu.sync_copy(o_ref, onesv)

        @pl.loop(0, S_NK)
        def _step(k):
            pltpu.sync_copy(onesv, acc.at[idxv.at[k]], add=True)

        plsc.subcore_barrier()

        @pl.when(sid == 0)
        def _flush():
            pltpu.sync_copy(acc, out_ref.at[cid])

    return pl.kernel(
        body,
        out_type=jax.ShapeDtypeStruct((2, N, 128), _f32),
        mesh=_sc_mesh(),
        scratch_types=[
            pltpu.VMEM((S_NK, S_CH), jnp.int32),
            pltpu.VMEM((S_CH, 128), _f32),
            pltpu.VMEM_SHARED((N, 128), _f32),
        ],
        name="sc_counts",
    )(col3, zeros, ones128)


def _tc_edge(gall, e_in, Ws, Wd, Wea, be, W1x, W1e, b1):
    EF = e_in.shape[1]

    def body(gs_ref, gd_ref, e_ref, Ws_ref, Wd_ref, Wea_ref, be_ref,
             W1x_ref, W1e_ref, b1_ref, eo_ref, m_ref):
        gs = gs_ref[...]
        t = jnp.dot(gs, Ws_ref[...], preferred_element_type=_f32)
        t = t + jnp.dot(gd_ref[...], Wd_ref[...], preferred_element_type=_f32)
        t = t + jnp.dot(e_ref[...], Wea_ref[...], preferred_element_type=_f32)
        e2 = _relu(t + be_ref[...])
        mm = jnp.dot(gs, W1x_ref[...], preferred_element_type=_f32)
        mm = mm + jnp.dot(e2, W1e_ref[...], preferred_element_type=_f32)
        eo_ref[...] = e2
        m_ref[...] = _relu(mm + b1_ref[...])

    full = lambda a: pl.BlockSpec(a.shape, lambda i: (0,) * a.ndim)
    return pl.pallas_call(
        body,
        grid=(NB_E,),
        in_specs=[
            pl.BlockSpec((BE, 128), lambda i: (i, 0)),
            pl.BlockSpec((BE, 128), lambda i: (i + NB_E, 0)),
            pl.BlockSpec((BE, EF), lambda i: (i, 0)),
            full(Ws), full(Wd), full(Wea), full(be), full(W1x), full(W1e), full(b1),
        ],
        out_specs=[
            pl.BlockSpec((BE, 256), lambda i: (i, 0)),
            pl.BlockSpec((BE, 128), lambda i: (i, 0)),
        ],
        out_shape=[
            jax.ShapeDtypeStruct((E, 256), _f32),
            jax.ShapeDtypeStruct((E, 128), _f32),
        ],
        name="tc_edge",
    )(gall, gall, e_in, Ws, Wd, Wea, be, W1x, W1e, b1)


def _tc_node(x, parts, cnt_parts, batch3, u, W2x, W2a, b2, Wgu, Wgn, bg):
    has_u = u is not None

    def body(*refs):
        if has_u:
            (x_ref, p_ref, c_ref, b_ref, u_ref, W2x_ref, W2a_ref, b2_ref,
             Wgu_ref, Wgn_ref, bg_ref, xo_ref, uo_ref, nagg_acc, ncnt_acc) = refs
        else:
            (x_ref, p_ref, c_ref, b_ref, W2x_ref, W2a_ref, b2_ref,
             Wgn_ref, bg_ref, xo_ref, uo_ref, nagg_acc, ncnt_acc) = refs
        i = pl.program_id(0)
        cnt = c_ref[0, :, 0:1] + c_ref[1, :, 0:1]
        agg = (p_ref[0] + p_ref[1]) / jnp.maximum(cnt, 1.0)
        xn = jnp.dot(x_ref[...], W2x_ref[...], preferred_element_type=_f32)
        xn = xn + jnp.dot(agg, W2a_ref[...], preferred_element_type=_f32)
        xn = _relu(xn + b2_ref[...])
        xo_ref[...] = xn

        bb = b_ref[0, 0, :]
        oh = (bb[:, None] == lax.broadcasted_iota(jnp.int32, (BN, G), 1))
        ohf = oh.astype(_f32)

        @pl.when(i == 0)
        def _init():
            nagg_acc[...] = jnp.zeros((G, 128), _f32)
            ncnt_acc[...] = jnp.zeros((G, 128), _f32)

        dn = (((0,), (0,)), ((), ()))
        nagg_acc[...] += lax.dot_general(ohf, xn, dn, preferred_element_type=_f32)
        ncnt_acc[...] += lax.dot_general(ohf, jnp.ones((BN, 128), _f32), dn,
                                         preferred_element_type=_f32)

        @pl.when(i == NB_N - 1)
        def _glob():
            nagg = nagg_acc[...] / jnp.maximum(ncnt_acc[...], 1.0)
            gg = jnp.dot(nagg, Wgn_ref[...], preferred_element_type=_f32)
            if has_u:
                gg = gg + jnp.dot(u_ref[...], Wgu_ref[...],
                                  preferred_element_type=_f32)
            uo_ref[...] = _relu(gg + bg_ref[...])

    full = lambda a: pl.BlockSpec(a.shape, lambda i: (0,) * a.ndim)
    ins = [x, parts, cnt_parts, batch3]
    in_specs = [
        pl.BlockSpec((BN, 128), lambda i: (i, 0)),
        pl.BlockSpec((2, BN, 128), lambda i: (0, i, 0)),
        pl.BlockSpec((2, BN, 128), lambda i: (0, i, 0)),
        pl.BlockSpec((1, 1, BN), lambda i: (i, 0, 0)),
    ]
    if has_u:
        ins += [u, W2x, W2a, b2, Wgu, Wgn, bg]
        in_specs += [full(u), full(W2x), full(W2a), full(b2), full(Wgu),
                     full(Wgn), full(bg)]
    else:
        ins += [W2x, W2a, b2, Wgn, bg]
        in_specs += [full(W2x), full(W2a), full(b2), full(Wgn), full(bg)]

    return pl.pallas_call(
        body,
        grid=(NB_N,),
        in_specs=in_specs,
        out_specs=[
            pl.BlockSpec((BN, 128), lambda i: (i, 0)),
            pl.BlockSpec((G, 128), lambda i: (0, 0)),
        ],
        out_shape=[
            jax.ShapeDtypeStruct((N, 128), _f32),
            jax.ShapeDtypeStruct((G, 128), _f32),
        ],
        scratch_shapes=[pltpu.VMEM((G, 128), _f32), pltpu.VMEM((G, 128), _f32)],
        name="tc_node",
    )(*ins)


def _tc_head(ur, up, Wr, br, Wp, bp):
    def body(ur_ref, up_ref, Wr_ref, br_ref, Wp_ref, bp_ref, o_ref):
        rg = jnp.dot(ur_ref[...], Wr_ref[...], preferred_element_type=_f32) + br_ref[...]
        pg = jnp.dot(up_ref[...], Wp_ref[...], preferred_element_type=_f32) + bp_ref[...]
        num = jnp.sum(rg * pg, axis=1, keepdims=True)
        nr = jnp.maximum(jnp.sqrt(jnp.sum(rg * rg, axis=1, keepdims=True)), 1e-8)
        npp = jnp.maximum(jnp.sqrt(jnp.sum(pg * pg, axis=1, keepdims=True)), 1e-8)
        o_ref[...] = jax.nn.sigmoid(num / (nr * npp)) * jnp.ones((1, 128), _f32)

    return pl.pallas_call(
        body,
        out_shape=jax.ShapeDtypeStruct((G, 128), _f32),
        name="tc_head",
    )(ur, up, Wr, br, Wp, bp)


def _row2(v):
    return v.reshape(1, -1)


def _chain(x, e, u, edge_index, batch, layer_params):
    idx3 = edge_index.reshape(NW, G_NK, G_CH)
    col3 = edge_index[1].reshape(NW, S_NK, S_CH)
    batch3 = batch.reshape(NB_N, 1, BN)
    zeros = jnp.zeros((N, 128), _f32)
    ones128 = jnp.ones((S_CH, 128), _f32)
    cnt_parts = _sc_counts(col3, zeros, ones128)

    for p in layer_params:
        We, be = p['edge']
        W1, b1 = p['node1']
        W2, b2 = p['node2']
        Wg, bg = p['glob']
        g = _sc_gather(x, idx3)
        e, m = _tc_edge(g, e, We[:128], We[128:256], We[256:], _row2(be),
                        W1[:128], W1[128:], _row2(b1))
        parts = _sc_scatter(m, col3, zeros)
        if u is None:
            Wgu, Wgn = None, Wg
        else:
            gu = u.shape[1]
            Wgu, Wgn = Wg[:gu], Wg[gu:]
        x, u = _tc_node(x, parts, cnt_parts, batch3, u,
                        W2[:128], W2[128:], _row2(b2), Wgu, Wgn, _row2(bg))
    return u


def kernel(rx, r_edge_attr, r_edge_index, r_batch, r_g, px, p_edge_attr,
           p_edge_index, p_batch, params):
    ur = _chain(rx, r_edge_attr, r_g, r_edge_index, r_batch,
                [params[k] for k in ('r1', 'r2', 'r3', 'r4')])
    up = _chain(px, p_edge_attr, None, p_edge_index, p_batch,
                [params[k] for k in ('p1', 'p2', 'p3', 'p4')])
    Wr, br = params['r_lin1']
    Wp, bp = params['p_lin1']
    out = _tc_head(ur, up, Wr, _row2(br), Wp, _row2(bp))
    return out[:, 0]


# speedup vs baseline: 3.0138x; 1.4200x over previous
import jax
import jax.numpy as jnp
from jax import lax
from jax.experimental import pallas as pl
from jax.experimental.pallas import tpu as pltpu
from jax.experimental.pallas import tpu_sc as plsc

N = 10000
E = 160000
G = 16
NF = 128

NCORES = 2
NSUB = 16
NW = NCORES * NSUB
SLAB = 624

G_RPW = 2 * E // NW
G_CH = 80
G_NK = G_RPW // G_CH
G_NB = 5

S_RPW = E // NW
S_CH = 40
S_NK = S_RPW // S_CH
S_NB = 5

BE = 2000
NB_E = E // BE
BN = 1000
NB_N = N // BN

_f32 = jnp.float32
_bf16 = jnp.bfloat16


def _relu(v):
    return jnp.maximum(v, 0.0)


def _sc_mesh():
    return plsc.VectorSubcoreMesh(core_axis_name="c", subcore_axis_name="s")


def _by_slab(sid, copy):
    @pl.when(sid < NSUB - 1)
    def _a():
        copy(sid * SLAB, SLAB)

    @pl.when(sid == NSUB - 1)
    def _b():
        copy((NSUB - 1) * SLAB, N - (NSUB - 1) * SLAB)


def _sc_gather(table, idx3):
    W = table.shape[1]
    dt = table.dtype

    def body(table_ref, idx_ref, out_ref, idxv, *bufs):
        rows = bufs[:G_NB]
        sems = bufs[G_NB:]
        cid = lax.axis_index("c")
        sid = lax.axis_index("s")
        wid = sid * NCORES + cid
        base = wid * G_RPW
        pltpu.sync_copy(idx_ref.at[wid], idxv)
        for b in range(G_NB):
            pltpu.async_copy(table_ref.at[idxv.at[b]], rows[b], sems[b])

        @pl.loop(0, G_NK, step=G_NB)
        def _step(k):
            for b in range(G_NB):
                pltpu.make_async_copy(
                    table_ref.at[idxv.at[k + b]], rows[b], sems[b]).wait()
                pltpu.sync_copy(rows[b],
                                out_ref.at[pl.ds(base + (k + b) * G_CH, G_CH)])

                @pl.when(k + b + G_NB < G_NK)
                def _nxt():
                    pltpu.async_copy(
                        table_ref.at[idxv.at[k + b + G_NB]], rows[b], sems[b])

    return pl.kernel(
        body,
        out_type=jax.ShapeDtypeStruct((2 * E, W), dt),
        mesh=_sc_mesh(),
        scratch_types=[pltpu.VMEM((G_NK, G_CH), jnp.int32)]
        + [pltpu.VMEM((G_CH, W), dt) for _ in range(G_NB)]
        + [pltpu.SemaphoreType.DMA for _ in range(G_NB)],
        name="sc_gather_x",
    )(table, idx3)


def _sc_scatter(m, col3, zeros):
    def body(m_ref, col_ref, z_ref, out_ref, idxv, *bufs):
        rows = bufs[:S_NB]
        sems = bufs[S_NB:-1]
        acc = bufs[-1]
        cid = lax.axis_index("c")
        sid = lax.axis_index("s")
        wid = sid * NCORES + cid
        base = wid * S_RPW
        _by_slab(sid, lambda off, sz: pltpu.sync_copy(
            z_ref.at[pl.ds(off, sz)], acc.at[pl.ds(off, sz)]))
        plsc.subcore_barrier()
        pltpu.sync_copy(col_ref.at[wid], idxv)
        for b in range(S_NB):
            pltpu.async_copy(m_ref.at[pl.ds(base + b * S_CH, S_CH)], rows[b], sems[b])

        @pl.loop(0, S_NK, step=S_NB)
        def _step(k):
            for b in range(S_NB):
                pltpu.make_async_copy(
                    m_ref.at[pl.ds(base + (k + b) * S_CH, S_CH)], rows[b], sems[b]).wait()
                pltpu.sync_copy(rows[b], acc.at[idxv.at[k + b]], add=True)

                @pl.when(k + b + S_NB < S_NK)
                def _nxt():
                    pltpu.async_copy(
                        m_ref.at[pl.ds(base + (k + b + S_NB) * S_CH, S_CH)],
                        rows[b], sems[b])

        plsc.subcore_barrier()
        _by_slab(sid, lambda off, sz: pltpu.sync_copy(
            acc.at[pl.ds(off, sz)], out_ref.at[cid, pl.ds(off, sz)]))

    return pl.kernel(
        body,
        out_type=jax.ShapeDtypeStruct((2, N, 128), _f32),
        mesh=_sc_mesh(),
        scratch_types=[pltpu.VMEM((S_NK, S_CH), jnp.int32)]
        + [pltpu.VMEM((S_CH, 128), _f32) for _ in range(S_NB)]
        + [pltpu.SemaphoreType.DMA for _ in range(S_NB)]
        + [pltpu.VMEM_SHARED((N, 128), _f32)],
        name="sc_scatter_m",
    )(m, col3, zeros)


def _sc_counts(col3, zeros, ones128):
    def body(col_ref, z_ref, o_ref, out_ref, idxv, onesv, acc):
        cid = lax.axis_index("c")
        sid = lax.axis_index("s")
        wid = sid * NCORES + cid
        _by_slab(sid, lambda off, sz: pltpu.sync_copy(
            z_ref.at[pl.ds(off, sz)], acc.at[pl.ds(off, sz)]))
        plsc.subcore_barrier()
        pltpu.sync_copy(col_ref.at[wid], idxv)
        pltpu.sync_copy(o_ref, onesv)

        @pl.loop(0, S_NK)
        def _step(k):
            pltpu.sync_copy(onesv, acc.at[idxv.at[k]], add=True)

        plsc.subcore_barrier()
        _by_slab(sid, lambda off, sz: pltpu.sync_copy(
            acc.at[pl.ds(off, sz)], out_ref.at[cid, pl.ds(off, sz)]))

    return pl.kernel(
        body,
        out_type=jax.ShapeDtypeStruct((2, N, 128), _f32),
        mesh=_sc_mesh(),
        scratch_types=[
            pltpu.VMEM((S_NK, S_CH), jnp.int32),
            pltpu.VMEM((S_CH, 128), _f32),
            pltpu.VMEM_SHARED((N, 128), _f32),
        ],
        name="sc_counts",
    )(col3, zeros, ones128)


def _tc_edge(gall, e_in, Ws, Wd, Wea, be, W1x, W1e, b1):
    EF = e_in.shape[1]

    def body(gs_ref, gd_ref, e_ref, Ws_ref, Wd_ref, Wea_ref, be_ref,
             W1x_ref, W1e_ref, b1_ref, eo_ref, m_ref):
        gs = gs_ref[...].astype(_bf16)
        gd = gd_ref[...].astype(_bf16)
        t = jnp.dot(gs, Ws_ref[...], preferred_element_type=_f32)
        t = t + jnp.dot(gd, Wd_ref[...], preferred_element_type=_f32)
        t = t + jnp.dot(e_ref[...], Wea_ref[...], preferred_element_type=_f32)
        e2 = _relu(t + be_ref[...])
        e2b = e2.astype(_bf16)
        mm = jnp.dot(gs, W1x_ref[...], preferred_element_type=_f32)
        mm = mm + jnp.dot(e2b, W1e_ref[...], preferred_element_type=_f32)
        eo_ref[...] = e2b
        m_ref[...] = _relu(mm + b1_ref[...])

    full = lambda a: pl.BlockSpec(a.shape, lambda i: (0,) * a.ndim)
    return pl.pallas_call(
        body,
        grid=(NB_E,),
        in_specs=[
            pl.BlockSpec((BE, 128), lambda i: (i, 0)),
            pl.BlockSpec((BE, 128), lambda i: (i + NB_E, 0)),
            pl.BlockSpec((BE, EF), lambda i: (i, 0)),
            full(Ws), full(Wd), full(Wea), full(be), full(W1x), full(W1e), full(b1),
        ],
        out_specs=[
            pl.BlockSpec((BE, 256), lambda i: (i, 0)),
            pl.BlockSpec((BE, 128), lambda i: (i, 0)),
        ],
        out_shape=[
            jax.ShapeDtypeStruct((E, 256), _bf16),
            jax.ShapeDtypeStruct((E, 128), _f32),
        ],
        name="tc_edge",
    )(gall, gall, e_in, Ws, Wd, Wea, be, W1x, W1e, b1)


def _tc_node(x, parts, cnt_parts, batch3, u, W2x, W2a, b2, Wgu, Wgn, bg):
    has_u = u is not None

    def body(*refs):
        if has_u:
            (x_ref, p_ref, c_ref, b_ref, u_ref, W2x_ref, W2a_ref, b2_ref,
             Wgu_ref, Wgn_ref, bg_ref, xo_ref, uo_ref, nagg_acc, ncnt_acc) = refs
        else:
            (x_ref, p_ref, c_ref, b_ref, W2x_ref, W2a_ref, b2_ref,
             Wgn_ref, bg_ref, xo_ref, uo_ref, nagg_acc, ncnt_acc) = refs
        i = pl.program_id(0)
        cnt = c_ref[0, :, 0:1] + c_ref[1, :, 0:1]
        agg = (p_ref[0] + p_ref[1]) / jnp.maximum(cnt, 1.0)
        xn = jnp.dot(x_ref[...], W2x_ref[...], preferred_element_type=_f32)
        xn = xn + jnp.dot(agg, W2a_ref[...], preferred_element_type=_f32)
        xn = _relu(xn + b2_ref[...])
        xo_ref[...] = xn

        bb = b_ref[0, 0, :]
        oh = (bb[:, None] == lax.broadcasted_iota(jnp.int32, (BN, G), 1))
        ohf = oh.astype(_f32)

        @pl.when(i == 0)
        def _init():
            nagg_acc[...] = jnp.zeros((G, 128), _f32)
            ncnt_acc[...] = jnp.zeros((G, 128), _f32)

        dn = (((0,), (0,)), ((), ()))
        nagg_acc[...] += lax.dot_general(ohf, xn, dn, preferred_element_type=_f32)
        ncnt_acc[...] += lax.dot_general(ohf, jnp.ones((BN, 128), _f32), dn,
                                         preferred_element_type=_f32)

        @pl.when(i == NB_N - 1)
        def _glob():
            nagg = nagg_acc[...] / jnp.maximum(ncnt_acc[...], 1.0)
            gg = jnp.dot(nagg, Wgn_ref[...], preferred_element_type=_f32)
            if has_u:
                gg = gg + jnp.dot(u_ref[...], Wgu_ref[...],
                                  preferred_element_type=_f32)
            uo_ref[...] = _relu(gg + bg_ref[...])

    full = lambda a: pl.BlockSpec(a.shape, lambda i: (0,) * a.ndim)
    ins = [x, parts, cnt_parts, batch3]
    in_specs = [
        pl.BlockSpec((BN, 128), lambda i: (i, 0)),
        pl.BlockSpec((2, BN, 128), lambda i: (0, i, 0)),
        pl.BlockSpec((2, BN, 128), lambda i: (0, i, 0)),
        pl.BlockSpec((1, 1, BN), lambda i: (i, 0, 0)),
    ]
    if has_u:
        ins += [u, W2x, W2a, b2, Wgu, Wgn, bg]
        in_specs += [full(u), full(W2x), full(W2a), full(b2), full(Wgu),
                     full(Wgn), full(bg)]
    else:
        ins += [W2x, W2a, b2, Wgn, bg]
        in_specs += [full(W2x), full(W2a), full(b2), full(Wgn), full(bg)]

    return pl.pallas_call(
        body,
        grid=(NB_N,),
        in_specs=in_specs,
        out_specs=[
            pl.BlockSpec((BN, 128), lambda i: (i, 0)),
            pl.BlockSpec((G, 128), lambda i: (0, 0)),
        ],
        out_shape=[
            jax.ShapeDtypeStruct((N, 128), _f32),
            jax.ShapeDtypeStruct((G, 128), _f32),
        ],
        scratch_shapes=[pltpu.VMEM((G, 128), _f32), pltpu.VMEM((G, 128), _f32)],
        name="tc_node",
    )(*ins)


def _tc_head(ur, up, Wr, br, Wp, bp):
    def body(ur_ref, up_ref, Wr_ref, br_ref, Wp_ref, bp_ref, o_ref):
        rg = jnp.dot(ur_ref[...], Wr_ref[...], preferred_element_type=_f32) + br_ref[...]
        pg = jnp.dot(up_ref[...], Wp_ref[...], preferred_element_type=_f32) + bp_ref[...]
        num = jnp.sum(rg * pg, axis=1, keepdims=True)
        nr = jnp.maximum(jnp.sqrt(jnp.sum(rg * rg, axis=1, keepdims=True)), 1e-8)
        npp = jnp.maximum(jnp.sqrt(jnp.sum(pg * pg, axis=1, keepdims=True)), 1e-8)
        o_ref[...] = jax.nn.sigmoid(num / (nr * npp)) * jnp.ones((1, 128), _f32)

    return pl.pallas_call(
        body,
        out_shape=jax.ShapeDtypeStruct((G, 128), _f32),
        name="tc_head",
    )(ur, up, Wr, br, Wp, bp)


def _row2(v):
    return v.reshape(1, -1)


def _chain(x, e, u, edge_index, batch, layer_params):
    idx3 = edge_index.reshape(NW, G_NK, G_CH)
    col3 = edge_index[1].reshape(NW, S_NK, S_CH)
    batch3 = batch.reshape(NB_N, 1, BN)
    zeros = jnp.zeros((N, 128), _f32)
    ones128 = jnp.ones((S_CH, 128), _f32)
    cnt_parts = _sc_counts(col3, zeros, ones128)
    e = e.astype(_bf16)

    for p in layer_params:
        We, be = p['edge']
        W1, b1 = p['node1']
        W2, b2 = p['node2']
        Wg, bg = p['glob']
        g = _sc_gather(x, idx3)
        e, m = _tc_edge(g, e,
                        We[:128].astype(_bf16), We[128:256].astype(_bf16),
                        We[256:].astype(_bf16), _row2(be),
                        W1[:128].astype(_bf16), W1[128:].astype(_bf16), _row2(b1))
        parts = _sc_scatter(m, col3, zeros)
        if u is None:
            Wgu, Wgn = None, Wg
        else:
            gu = u.shape[1]
            Wgu, Wgn = Wg[:gu], Wg[gu:]
        x, u = _tc_node(x, parts, cnt_parts, batch3, u,
                        W2[:128], W2[128:], _row2(b2), Wgu, Wgn, _row2(bg))
    return u


def kernel(rx, r_edge_attr, r_edge_index, r_batch, r_g, px, p_edge_attr,
           p_edge_index, p_batch, params):
    ur = _chain(rx, r_edge_attr, r_g, r_edge_index, r_batch,
                [params[k] for k in ('r1', 'r2', 'r3', 'r4')])
    up = _chain(px, p_edge_attr, None, p_edge_index, p_batch,
                [params[k] for k in ('p1', 'p2', 'p3', 'p4')])
    Wr, br = params['r_lin1']
    Wp, bp = params['p_lin1']
    out = _tc_head(ur, up, Wr, _row2(br), Wp, _row2(bp))
    return out[:, 0]


# speedup vs baseline: 3.4728x; 1.1523x over previous
import jax
import jax.numpy as jnp
from jax import lax
from jax.experimental import pallas as pl
from jax.experimental.pallas import tpu as pltpu
from jax.experimental.pallas import tpu_sc as plsc

N = 10000
E = 160000
G = 16
NF = 128

NCORES = 2
NSUB = 16
NW = NCORES * NSUB
SLAB = 624

G_RPW = 2 * E // NW
G_CH = 40
G_NK = G_RPW // G_CH
G_NB = 2

S_RPW = E // NW
S_CH = 40
S_NK = S_RPW // S_CH
S_NB = 5

BE = 2000
NB_E = E // BE
BN = 1000
NB_N = N // BN

_f32 = jnp.float32
_bf16 = jnp.bfloat16


def _relu(v):
    return jnp.maximum(v, 0.0)


def _sc_mesh():
    return plsc.VectorSubcoreMesh(core_axis_name="c", subcore_axis_name="s")


def _by_slab(sid, copy):
    @pl.when(sid < NSUB - 1)
    def _a():
        copy(sid * SLAB, SLAB)

    @pl.when(sid == NSUB - 1)
    def _b():
        copy((NSUB - 1) * SLAB, N - (NSUB - 1) * SLAB)


def _sc_gather(table, idx3):
    W = table.shape[1]
    dt = table.dtype

    def body(table_ref, idx_ref, out_ref, idxv, tshared, *bufs):
        rows = bufs[:G_NB]
        gsems = bufs[G_NB:2 * G_NB]
        wsems = bufs[2 * G_NB:]
        cid = lax.axis_index("c")
        sid = lax.axis_index("s")
        wid = sid * NCORES + cid
        base = wid * G_RPW
        _by_slab(sid, lambda off, sz: pltpu.sync_copy(
            table_ref.at[pl.ds(off, sz)], tshared.at[pl.ds(off, sz)]))
        pltpu.sync_copy(idx_ref.at[wid], idxv)
        plsc.subcore_barrier()
        for b in range(G_NB):
            pltpu.async_copy(tshared.at[idxv.at[b]], rows[b], gsems[b])

        @pl.loop(0, G_NK, step=G_NB)
        def _step(k):
            for b in range(G_NB):
                pltpu.make_async_copy(
                    tshared.at[idxv.at[k + b]], rows[b], gsems[b]).wait()
                dst = out_ref.at[pl.ds(base + (k + b) * G_CH, G_CH)]
                pltpu.async_copy(rows[b], dst, wsems[b])

                @pl.when(k + b + G_NB < G_NK)
                def _nxt():
                    pltpu.make_async_copy(rows[b], dst, wsems[b]).wait()
                    pltpu.async_copy(
                        tshared.at[idxv.at[k + b + G_NB]], rows[b], gsems[b])

        for b in range(G_NB):
            pltpu.make_async_copy(
                rows[b],
                out_ref.at[pl.ds(base + (G_NK - G_NB + b) * G_CH, G_CH)],
                wsems[b]).wait()

    return pl.kernel(
        body,
        out_type=jax.ShapeDtypeStruct((2 * E, W), dt),
        mesh=_sc_mesh(),
        scratch_types=[pltpu.VMEM((G_NK, G_CH), jnp.int32),
                       pltpu.VMEM_SHARED((N, 128), dt)]
        + [pltpu.VMEM((G_CH, W), dt) for _ in range(G_NB)]
        + [pltpu.SemaphoreType.DMA for _ in range(2 * G_NB)],
        name="sc_gather_x",
    )(table, idx3)


def _sc_scatter(m, col3, zeros):
    def body(m_ref, col_ref, z_ref, out_ref, idxv, *bufs):
        rows = bufs[:S_NB]
        sems = bufs[S_NB:-1]
        acc = bufs[-1]
        cid = lax.axis_index("c")
        sid = lax.axis_index("s")
        wid = sid * NCORES + cid
        base = wid * S_RPW
        _by_slab(sid, lambda off, sz: pltpu.sync_copy(
            z_ref.at[pl.ds(off, sz)], acc.at[pl.ds(off, sz)]))
        plsc.subcore_barrier()
        pltpu.sync_copy(col_ref.at[wid], idxv)
        for b in range(S_NB):
            pltpu.async_copy(m_ref.at[pl.ds(base + b * S_CH, S_CH)], rows[b], sems[b])

        @pl.loop(0, S_NK, step=S_NB)
        def _step(k):
            for b in range(S_NB):
                pltpu.make_async_copy(
                    m_ref.at[pl.ds(base + (k + b) * S_CH, S_CH)], rows[b], sems[b]).wait()
                pltpu.sync_copy(rows[b], acc.at[idxv.at[k + b]], add=True)

                @pl.when(k + b + S_NB < S_NK)
                def _nxt():
                    pltpu.async_copy(
                        m_ref.at[pl.ds(base + (k + b + S_NB) * S_CH, S_CH)],
                        rows[b], sems[b])

        plsc.subcore_barrier()
        _by_slab(sid, lambda off, sz: pltpu.sync_copy(
            acc.at[pl.ds(off, sz)], out_ref.at[cid, pl.ds(off, sz)]))

    return pl.kernel(
        body,
        out_type=jax.ShapeDtypeStruct((2, N, 128), _f32),
        mesh=_sc_mesh(),
        scratch_types=[pltpu.VMEM((S_NK, S_CH), jnp.int32)]
        + [pltpu.VMEM((S_CH, 128), _f32) for _ in range(S_NB)]
        + [pltpu.SemaphoreType.DMA for _ in range(S_NB)]
        + [pltpu.VMEM_SHARED((N, 128), _f32)],
        name="sc_scatter_m",
    )(m, col3, zeros)


def _sc_counts(col3, zeros, ones128):
    def body(col_ref, z_ref, o_ref, out_ref, idxv, onesv, acc):
        cid = lax.axis_index("c")
        sid = lax.axis_index("s")
        wid = sid * NCORES + cid
        _by_slab(sid, lambda off, sz: pltpu.sync_copy(
            z_ref.at[pl.ds(off, sz)], acc.at[pl.ds(off, sz)]))
        plsc.subcore_barrier()
        pltpu.sync_copy(col_ref.at[wid], idxv)
        pltpu.sync_copy(o_ref, onesv)

        @pl.loop(0, S_NK)
        def _step(k):
            pltpu.sync_copy(onesv, acc.at[idxv.at[k]], add=True)

        plsc.subcore_barrier()
        _by_slab(sid, lambda off, sz: pltpu.sync_copy(
            acc.at[pl.ds(off, sz)], out_ref.at[cid, pl.ds(off, sz)]))

    return pl.kernel(
        body,
        out_type=jax.ShapeDtypeStruct((2, N, 128), _f32),
        mesh=_sc_mesh(),
        scratch_types=[
            pltpu.VMEM((S_NK, S_CH), jnp.int32),
            pltpu.VMEM((S_CH, 128), _f32),
            pltpu.VMEM_SHARED((N, 128), _f32),
        ],
        name="sc_counts",
    )(col3, zeros, ones128)


def _tc_edge(gall, e_in, Ws, Wd, Wea, be, W1x, W1e, b1):
    EF = e_in.shape[1]

    def body(gs_ref, gd_ref, e_ref, Ws_ref, Wd_ref, Wea_ref, be_ref,
             W1x_ref, W1e_ref, b1_ref, eo_ref, m_ref):
        gs = gs_ref[...].astype(_bf16)
        gd = gd_ref[...].astype(_bf16)
        t = jnp.dot(gs, Ws_ref[...], preferred_element_type=_f32)
        t = t + jnp.dot(gd, Wd_ref[...], preferred_element_type=_f32)
        t = t + jnp.dot(e_ref[...], Wea_ref[...], preferred_element_type=_f32)
        e2 = _relu(t + be_ref[...])
        e2b = e2.astype(_bf16)
        mm = jnp.dot(gs, W1x_ref[...], preferred_element_type=_f32)
        mm = mm + jnp.dot(e2b, W1e_ref[...], preferred_element_type=_f32)
        eo_ref[...] = e2b
        m_ref[...] = _relu(mm + b1_ref[...])

    full = lambda a: pl.BlockSpec(a.shape, lambda i: (0,) * a.ndim)
    return pl.pallas_call(
        body,
        grid=(NB_E,),
        in_specs=[
            pl.BlockSpec((BE, 128), lambda i: (i, 0)),
            pl.BlockSpec((BE, 128), lambda i: (i + NB_E, 0)),
            pl.BlockSpec((BE, EF), lambda i: (i, 0)),
            full(Ws), full(Wd), full(Wea), full(be), full(W1x), full(W1e), full(b1),
        ],
        out_specs=[
            pl.BlockSpec((BE, 256), lambda i: (i, 0)),
            pl.BlockSpec((BE, 128), lambda i: (i, 0)),
        ],
        out_shape=[
            jax.ShapeDtypeStruct((E, 256), _bf16),
            jax.ShapeDtypeStruct((E, 128), _f32),
        ],
        name="tc_edge",
    )(gall, gall, e_in, Ws, Wd, Wea, be, W1x, W1e, b1)


def _tc_node(x, parts, cnt_parts, batch3, u, W2x, W2a, b2, Wgu, Wgn, bg):
    has_u = u is not None

    def body(*refs):
        if has_u:
            (x_ref, p_ref, c_ref, b_ref, u_ref, W2x_ref, W2a_ref, b2_ref,
             Wgu_ref, Wgn_ref, bg_ref, xo_ref, uo_ref, nagg_acc, ncnt_acc) = refs
        else:
            (x_ref, p_ref, c_ref, b_ref, W2x_ref, W2a_ref, b2_ref,
             Wgn_ref, bg_ref, xo_ref, uo_ref, nagg_acc, ncnt_acc) = refs
        i = pl.program_id(0)
        cnt = c_ref[0, :, 0:1] + c_ref[1, :, 0:1]
        agg = (p_ref[0] + p_ref[1]) / jnp.maximum(cnt, 1.0)
        xn = jnp.dot(x_ref[...], W2x_ref[...], preferred_element_type=_f32)
        xn = xn + jnp.dot(agg, W2a_ref[...], preferred_element_type=_f32)
        xn = _relu(xn + b2_ref[...])
        xo_ref[...] = xn

        bb = b_ref[0, 0, :]
        oh = (bb[:, None] == lax.broadcasted_iota(jnp.int32, (BN, G), 1))
        ohf = oh.astype(_f32)

        @pl.when(i == 0)
        def _init():
            nagg_acc[...] = jnp.zeros((G, 128), _f32)
            ncnt_acc[...] = jnp.zeros((G, 128), _f32)

        dn = (((0,), (0,)), ((), ()))
        nagg_acc[...] += lax.dot_general(ohf, xn, dn, preferred_element_type=_f32)
        ncnt_acc[...] += lax.dot_general(ohf, jnp.ones((BN, 128), _f32), dn,
                                         preferred_element_type=_f32)

        @pl.when(i == NB_N - 1)
        def _glob():
            nagg = nagg_acc[...] / jnp.maximum(ncnt_acc[...], 1.0)
            gg = jnp.dot(nagg, Wgn_ref[...], preferred_element_type=_f32)
            if has_u:
                gg = gg + jnp.dot(u_ref[...], Wgu_ref[...],
                                  preferred_element_type=_f32)
            uo_ref[...] = _relu(gg + bg_ref[...])

    full = lambda a: pl.BlockSpec(a.shape, lambda i: (0,) * a.ndim)
    ins = [x, parts, cnt_parts, batch3]
    in_specs = [
        pl.BlockSpec((BN, 128), lambda i: (i, 0)),
        pl.BlockSpec((2, BN, 128), lambda i: (0, i, 0)),
        pl.BlockSpec((2, BN, 128), lambda i: (0, i, 0)),
        pl.BlockSpec((1, 1, BN), lambda i: (i, 0, 0)),
    ]
    if has_u:
        ins += [u, W2x, W2a, b2, Wgu, Wgn, bg]
        in_specs += [full(u), full(W2x), full(W2a), full(b2), full(Wgu),
                     full(Wgn), full(bg)]
    else:
        ins += [W2x, W2a, b2, Wgn, bg]
        in_specs += [full(W2x), full(W2a), full(b2), full(Wgn), full(bg)]

    return pl.pallas_call(
        body,
        grid=(NB_N,),
        in_specs=in_specs,
        out_specs=[
            pl.BlockSpec((BN, 128), lambda i: (i, 0)),
            pl.BlockSpec((G, 128), lambda i: (0, 0)),
        ],
        out_shape=[
            jax.ShapeDtypeStruct((N, 128), _f32),
            jax.ShapeDtypeStruct((G, 128), _f32),
        ],
        scratch_shapes=[pltpu.VMEM((G, 128), _f32), pltpu.VMEM((G, 128), _f32)],
        name="tc_node",
    )(*ins)


def _tc_head(ur, up, Wr, br, Wp, bp):
    def body(ur_ref, up_ref, Wr_ref, br_ref, Wp_ref, bp_ref, o_ref):
        rg = jnp.dot(ur_ref[...], Wr_ref[...], preferred_element_type=_f32) + br_ref[...]
        pg = jnp.dot(up_ref[...], Wp_ref[...], preferred_element_type=_f32) + bp_ref[...]
        num = jnp.sum(rg * pg, axis=1, keepdims=True)
        nr = jnp.maximum(jnp.sqrt(jnp.sum(rg * rg, axis=1, keepdims=True)), 1e-8)
        npp = jnp.maximum(jnp.sqrt(jnp.sum(pg * pg, axis=1, keepdims=True)), 1e-8)
        o_ref[...] = jax.nn.sigmoid(num / (nr * npp)) * jnp.ones((1, 128), _f32)

    return pl.pallas_call(
        body,
        out_shape=jax.ShapeDtypeStruct((G, 128), _f32),
        name="tc_head",
    )(ur, up, Wr, br, Wp, bp)


def _row2(v):
    return v.reshape(1, -1)


class _Chain:
    def __init__(self, x, e, u, edge_index, batch, layer_params, zeros, ones128):
        self.x = x
        self.e = e.astype(_bf16)
        self.u = u
        self.idx3 = edge_index.reshape(NW, G_NK, G_CH)
        self.col3 = edge_index[1].reshape(NW, S_NK, S_CH)
        self.batch3 = batch.reshape(NB_N, 1, BN)
        self.zeros = zeros
        self.layers = layer_params
        self.cnt_parts = _sc_counts(self.col3, zeros, ones128)
        self.g = None
        self.m = None
        self.parts = None

    def gather(self):
        self.g = _sc_gather(self.x, self.idx3)

    def edge(self, k):
        We, be = self.layers[k]['edge']
        W1, b1 = self.layers[k]['node1']
        self.e, self.m = _tc_edge(
            self.g, self.e,
            We[:128].astype(_bf16), We[128:256].astype(_bf16),
            We[256:].astype(_bf16), _row2(be),
            W1[:128].astype(_bf16), W1[128:].astype(_bf16), _row2(b1))

    def scatter(self):
        self.parts = _sc_scatter(self.m, self.col3, self.zeros)

    def node(self, k):
        W2, b2 = self.layers[k]['node2']
        Wg, bg = self.layers[k]['glob']
        if self.u is None:
            Wgu, Wgn = None, Wg
        else:
            gu = self.u.shape[1]
            Wgu, Wgn = Wg[:gu], Wg[gu:]
        self.x, self.u = _tc_node(self.x, self.parts, self.cnt_parts,
                                  self.batch3, self.u,
                                  W2[:128], W2[128:], _row2(b2),
                                  Wgu, Wgn, _row2(bg))


def kernel(rx, r_edge_attr, r_edge_index, r_batch, r_g, px, p_edge_attr,
           p_edge_index, p_batch, params):
    zeros = jnp.zeros((N, 128), _f32)
    ones128 = jnp.ones((S_CH, 128), _f32)
    cr = _Chain(rx, r_edge_attr, r_g, r_edge_index, r_batch,
                [params[k] for k in ('r1', 'r2', 'r3', 'r4')], zeros, ones128)
    cp = _Chain(px, p_edge_attr, None, p_edge_index, p_batch,
                [params[k] for k in ('p1', 'p2', 'p3', 'p4')], zeros, ones128)
    for k in range(4):
        cr.gather()
        cp.gather()
        cr.edge(k)
        cr.scatter()
        cp.edge(k)
        cp.scatter()
        cr.node(k)
        cp.node(k)
    Wr, br = params['r_lin1']
    Wp, bp = params['p_lin1']
    out = _tc_head(cr.u, cp.u, Wr, _row2(br), Wp, _row2(bp))
    return out[:, 0]


# speedup vs baseline: 3.4795x; 1.0019x over previous
import jax
import jax.numpy as jnp
from jax import lax
from jax.experimental import pallas as pl
from jax.experimental.pallas import tpu as pltpu
from jax.experimental.pallas import tpu_sc as plsc

N = 10000
E = 160000
G = 16
NF = 128

NCORES = 2
NSUB = 16
NW = NCORES * NSUB
SLAB = 624

G_RPW = 2 * E // NW
G_CH = 40
G_NK = G_RPW // G_CH
G_NB = 2

S_RPW = E // NW
S_CH = 40
S_NK = S_RPW // S_CH
S_NB = 5

BE = 2000
NB_E = E // BE
BN = 1000
NB_N = N // BN

_f32 = jnp.float32
_bf16 = jnp.bfloat16


def _relu(v):
    return jnp.maximum(v, 0.0)


def _sc_mesh():
    return plsc.VectorSubcoreMesh(core_axis_name="c", subcore_axis_name="s")


def _by_slab(sid, copy):
    @pl.when(sid < NSUB - 1)
    def _a():
        copy(sid * SLAB, SLAB)

    @pl.when(sid == NSUB - 1)
    def _b():
        copy((NSUB - 1) * SLAB, N - (NSUB - 1) * SLAB)


def _sc_gather(table, idx3):
    def body(table_ref, idx_ref, out_ref, idxv, tshared, *bufs):
        rows = bufs[:G_NB]
        gsems = bufs[G_NB:2 * G_NB]
        wsems = bufs[2 * G_NB:]
        cid = lax.axis_index("c")
        sid = lax.axis_index("s")
        wid = sid * NCORES + cid
        base = wid * G_RPW
        _by_slab(sid, lambda off, sz: pltpu.sync_copy(
            table_ref.at[pl.ds(off, sz)], tshared.at[pl.ds(off, sz)]))
        pltpu.sync_copy(idx_ref.at[wid], idxv)
        plsc.subcore_barrier()
        for b in range(G_NB):
            pltpu.async_copy(tshared.at[idxv.at[b]], rows[b], gsems[b])

        @pl.loop(0, G_NK, step=G_NB)
        def _step(k):
            for b in range(G_NB):
                pltpu.make_async_copy(
                    tshared.at[idxv.at[k + b]], rows[b], gsems[b]).wait()
                dst = out_ref.at[pl.ds(base + (k + b) * G_CH, G_CH)]
                pltpu.async_copy(rows[b], dst, wsems[b])

                @pl.when(k + b + G_NB < G_NK)
                def _nxt():
                    pltpu.make_async_copy(rows[b], dst, wsems[b]).wait()
                    pltpu.async_copy(
                        tshared.at[idxv.at[k + b + G_NB]], rows[b], gsems[b])

        for b in range(G_NB):
            pltpu.make_async_copy(
                rows[b],
                out_ref.at[pl.ds(base + (G_NK - G_NB + b) * G_CH, G_CH)],
                wsems[b]).wait()

    return pl.kernel(
        body,
        out_type=jax.ShapeDtypeStruct((2 * E, 128), _f32),
        mesh=_sc_mesh(),
        scratch_types=[pltpu.VMEM((G_NK, G_CH), jnp.int32),
                       pltpu.VMEM_SHARED((N, 128), _f32)]
        + [pltpu.VMEM((G_CH, 128), _f32) for _ in range(G_NB)]
        + [pltpu.SemaphoreType.DMA for _ in range(2 * G_NB)],
        name="sc_gather_x",
    )(table, idx3)


def _sc_scatter(m, col3, zeros):
    def body(m_ref, col_ref, z_ref, out_ref, idxv, *bufs):
        rows = bufs[:S_NB]
        sems = bufs[S_NB:-1]
        acc = bufs[-1]
        cid = lax.axis_index("c")
        sid = lax.axis_index("s")
        wid = sid * NCORES + cid
        base = wid * S_RPW
        _by_slab(sid, lambda off, sz: pltpu.sync_copy(
            z_ref.at[pl.ds(off, sz)], acc.at[pl.ds(off, sz)]))
        plsc.subcore_barrier()
        pltpu.sync_copy(col_ref.at[wid], idxv)
        for b in range(S_NB):
            pltpu.async_copy(m_ref.at[pl.ds(base + b * S_CH, S_CH)], rows[b], sems[b])

        @pl.loop(0, S_NK, step=S_NB)
        def _step(k):
            for b in range(S_NB):
                pltpu.make_async_copy(
                    m_ref.at[pl.ds(base + (k + b) * S_CH, S_CH)], rows[b], sems[b]).wait()
                pltpu.sync_copy(rows[b], acc.at[idxv.at[k + b]], add=True)

                @pl.when(k + b + S_NB < S_NK)
                def _nxt():
                    pltpu.async_copy(
                        m_ref.at[pl.ds(base + (k + b + S_NB) * S_CH, S_CH)],
                        rows[b], sems[b])

        plsc.subcore_barrier()
        _by_slab(sid, lambda off, sz: pltpu.sync_copy(
            acc.at[pl.ds(off, sz)], out_ref.at[cid, pl.ds(off, sz)]))

    return pl.kernel(
        body,
        out_type=jax.ShapeDtypeStruct((2, N, 128), _f32),
        mesh=_sc_mesh(),
        scratch_types=[pltpu.VMEM((S_NK, S_CH), jnp.int32)]
        + [pltpu.VMEM((S_CH, 128), _f32) for _ in range(S_NB)]
        + [pltpu.SemaphoreType.DMA for _ in range(S_NB)]
        + [pltpu.VMEM_SHARED((N, 128), _f32)],
        name="sc_scatter_m",
    )(m, col3, zeros)


def _sc_counts(col3, zeros, ones128):
    def body(col_ref, z_ref, o_ref, out_ref, idxv, onesv, acc):
        cid = lax.axis_index("c")
        sid = lax.axis_index("s")
        wid = sid * NCORES + cid
        _by_slab(sid, lambda off, sz: pltpu.sync_copy(
            z_ref.at[pl.ds(off, sz)], acc.at[pl.ds(off, sz)]))
        plsc.subcore_barrier()
        pltpu.sync_copy(col_ref.at[wid], idxv)
        pltpu.sync_copy(o_ref, onesv)

        @pl.loop(0, S_NK)
        def _step(k):
            pltpu.sync_copy(onesv, acc.at[idxv.at[k]], add=True)

        plsc.subcore_barrier()
        _by_slab(sid, lambda off, sz: pltpu.sync_copy(
            acc.at[pl.ds(off, sz)], out_ref.at[cid, pl.ds(off, sz)]))

    return pl.kernel(
        body,
        out_type=jax.ShapeDtypeStruct((2, N, 128), _f32),
        mesh=_sc_mesh(),
        scratch_types=[
            pltpu.VMEM((S_NK, S_CH), jnp.int32),
            pltpu.VMEM((S_CH, 128), _f32),
            pltpu.VMEM_SHARED((N, 128), _f32),
        ],
        name="sc_counts",
    )(col3, zeros, ones128)


def _tc_edge(gall, e_in, Ws, Wd, Wea, be, W1x, W1e, b1):
    EF = e_in.shape[1]

    def body(gs_ref, gd_ref, e_ref, Ws_ref, Wd_ref, Wea_ref, be_ref,
             W1x_ref, W1e_ref, b1_ref, eo_ref, m_ref):
        gs = gs_ref[...].astype(_bf16)
        gd = gd_ref[...].astype(_bf16)
        t = jnp.dot(gs, Ws_ref[...], preferred_element_type=_f32)
        t = t + jnp.dot(gd, Wd_ref[...], preferred_element_type=_f32)
        t = t + jnp.dot(e_ref[...], Wea_ref[...], preferred_element_type=_f32)
        e2 = _relu(t + be_ref[...])
        e2b = e2.astype(_bf16)
        mm = jnp.dot(gs, W1x_ref[...], preferred_element_type=_f32)
        mm = mm + jnp.dot(e2b, W1e_ref[...], preferred_element_type=_f32)
        eo_ref[...] = e2b
        m_ref[...] = _relu(mm + b1_ref[...])

    full = lambda a: pl.BlockSpec(a.shape, lambda i: (0,) * a.ndim)
    return pl.pallas_call(
        body,
        grid=(NB_E,),
        in_specs=[
            pl.BlockSpec((BE, 128), lambda i: (i, 0)),
            pl.BlockSpec((BE, 128), lambda i: (i + NB_E, 0)),
            pl.BlockSpec((BE, EF), lambda i: (i, 0)),
            full(Ws), full(Wd), full(Wea), full(be), full(W1x), full(W1e), full(b1),
        ],
        out_specs=[
            pl.BlockSpec((BE, 256), lambda i: (i, 0)),
            pl.BlockSpec((BE, 128), lambda i: (i, 0)),
        ],
        out_shape=[
            jax.ShapeDtypeStruct((E, 256), _bf16),
            jax.ShapeDtypeStruct((E, 128), _f32),
        ],
        name="tc_edge",
    )(gall, gall, e_in, Ws, Wd, Wea, be, W1x, W1e, b1)


def _tc_node(x, parts, cnt_parts, batch3, u, W2x, W2a, b2, Wgu, Wgn, bg):
    has_u = u is not None

    def body(*refs):
        if has_u:
            (x_ref, p_ref, c_ref, b_ref, u_ref, W2x_ref, W2a_ref, b2_ref,
             Wgu_ref, Wgn_ref, bg_ref, xo_ref, uo_ref, nagg_acc, ncnt_acc) = refs
        else:
            (x_ref, p_ref, c_ref, b_ref, W2x_ref, W2a_ref, b2_ref,
             Wgn_ref, bg_ref, xo_ref, uo_ref, nagg_acc, ncnt_acc) = refs
        i = pl.program_id(0)
        cnt = c_ref[0, :, 0:1] + c_ref[1, :, 0:1]
        agg = (p_ref[0] + p_ref[1]) / jnp.maximum(cnt, 1.0)
        xn = jnp.dot(x_ref[...], W2x_ref[...], preferred_element_type=_f32)
        xn = xn + jnp.dot(agg, W2a_ref[...], preferred_element_type=_f32)
        xn = _relu(xn + b2_ref[...])
        xo_ref[...] = xn

        bb = b_ref[0, 0, :]
        oh = (bb[:, None] == lax.broadcasted_iota(jnp.int32, (BN, G), 1))
        ohf = oh.astype(_f32)

        @pl.when(i == 0)
        def _init():
            nagg_acc[...] = jnp.zeros((G, 128), _f32)
            ncnt_acc[...] = jnp.zeros((G, 128), _f32)

        dn = (((0,), (0,)), ((), ()))
        nagg_acc[...] += lax.dot_general(ohf, xn, dn, preferred_element_type=_f32)
        ncnt_acc[...] += lax.dot_general(ohf, jnp.ones((BN, 128), _f32), dn,
                                         preferred_element_type=_f32)

        @pl.when(i == NB_N - 1)
        def _glob():
            nagg = nagg_acc[...] / jnp.maximum(ncnt_acc[...], 1.0)
            gg = jnp.dot(nagg, Wgn_ref[...], preferred_element_type=_f32)
            if has_u:
                gg = gg + jnp.dot(u_ref[...], Wgu_ref[...],
                                  preferred_element_type=_f32)
            uo_ref[...] = _relu(gg + bg_ref[...])

    full = lambda a: pl.BlockSpec(a.shape, lambda i: (0,) * a.ndim)
    ins = [x, parts, cnt_parts, batch3]
    in_specs = [
        pl.BlockSpec((BN, 128), lambda i: (i, 0)),
        pl.BlockSpec((2, BN, 128), lambda i: (0, i, 0)),
        pl.BlockSpec((2, BN, 128), lambda i: (0, i, 0)),
        pl.BlockSpec((1, 1, BN), lambda i: (i, 0, 0)),
    ]
    if has_u:
        ins += [u, W2x, W2a, b2, Wgu, Wgn, bg]
        in_specs += [full(u), full(W2x), full(W2a), full(b2), full(Wgu),
                     full(Wgn), full(bg)]
    else:
        ins += [W2x, W2a, b2, Wgn, bg]
        in_specs += [full(W2x), full(W2a), full(b2), full(Wgn), full(bg)]

    return pl.pallas_call(
        body,
        grid=(NB_N,),
        in_specs=in_specs,
        out_specs=[
            pl.BlockSpec((BN, 128), lambda i: (i, 0)),
            pl.BlockSpec((G, 128), lambda i: (0, 0)),
        ],
        out_shape=[
            jax.ShapeDtypeStruct((N, 128), _f32),
            jax.ShapeDtypeStruct((G, 128), _f32),
        ],
        scratch_shapes=[pltpu.VMEM((G, 128), _f32), pltpu.VMEM((G, 128), _f32)],
        name="tc_node",
    )(*ins)


def _tc_head(ur, up, Wr, br, Wp, bp):
    def body(ur_ref, up_ref, Wr_ref, br_ref, Wp_ref, bp_ref, o_ref):
        rg = jnp.dot(ur_ref[...], Wr_ref[...], preferred_element_type=_f32) + br_ref[...]
        pg = jnp.dot(up_ref[...], Wp_ref[...], preferred_element_type=_f32) + bp_ref[...]
        num = jnp.sum(rg * pg, axis=1, keepdims=True)
        nr = jnp.maximum(jnp.sqrt(jnp.sum(rg * rg, axis=1, keepdims=True)), 1e-8)
        npp = jnp.maximum(jnp.sqrt(jnp.sum(pg * pg, axis=1, keepdims=True)), 1e-8)
        o_ref[...] = jax.nn.sigmoid(num / (nr * npp)) * jnp.ones((1, 128), _f32)

    return pl.pallas_call(
        body,
        out_shape=jax.ShapeDtypeStruct((G, 128), _f32),
        name="tc_head",
    )(ur, up, Wr, br, Wp, bp)


def _row2(v):
    return v.reshape(1, -1)


class _Chain:
    def __init__(self, x, e, u, edge_index, batch, layer_params, zeros, ones128):
        self.x = x
        self.e = e.astype(_bf16)
        self.u = u
        self.idx3 = edge_index.reshape(NW, G_NK, G_CH)
        self.col3 = edge_index[1].reshape(NW, S_NK, S_CH)
        self.batch3 = batch.reshape(NB_N, 1, BN)
        self.zeros = zeros
        self.layers = layer_params
        self.cnt_parts = _sc_counts(self.col3, zeros, ones128)
        self.g = None
        self.m = None
        self.parts = None

    def gather(self):
        self.g = _sc_gather(self.x, self.idx3)

    def edge(self, k):
        We, be = self.layers[k]['edge']
        W1, b1 = self.layers[k]['node1']
        self.e, self.m = _tc_edge(
            self.g, self.e,
            We[:128].astype(_bf16), We[128:256].astype(_bf16),
            We[256:].astype(_bf16), _row2(be),
            W1[:128].astype(_bf16), W1[128:].astype(_bf16), _row2(b1))

    def scatter(self):
        self.parts = _sc_scatter(self.m, self.col3, self.zeros)

    def node(self, k):
        W2, b2 = self.layers[k]['node2']
        Wg, bg = self.layers[k]['glob']
        if self.u is None:
            Wgu, Wgn = None, Wg
        else:
            gu = self.u.shape[1]
            Wgu, Wgn = Wg[:gu], Wg[gu:]
        self.x, self.u = _tc_node(self.x, self.parts, self.cnt_parts,
                                  self.batch3, self.u,
                                  W2[:128], W2[128:], _row2(b2),
                                  Wgu, Wgn, _row2(bg))


def kernel(rx, r_edge_attr, r_edge_index, r_batch, r_g, px, p_edge_attr,
           p_edge_index, p_batch, params):
    zeros = jnp.zeros((N, 128), _f32)
    ones128 = jnp.ones((S_CH, 128), _f32)
    cr = _Chain(rx, r_edge_attr, r_g, r_edge_index, r_batch,
                [params[k] for k in ('r1', 'r2', 'r3', 'r4')], zeros, ones128)
    cp = _Chain(px, p_edge_attr, None, p_edge_index, p_batch,
                [params[k] for k in ('p1', 'p2', 'p3', 'p4')], zeros, ones128)
    for k in range(4):
        cr.gather()
        cp.gather()
        cr.edge(k)
        cr.scatter()
        cp.edge(k)
        cp.scatter()
        cr.node(k)
        cp.node(k)
    Wr, br = params['r_lin1']
    Wp, bp = params['p_lin1']
    out = _tc_head(cr.u, cp.u, Wr, _row2(br), Wp, _row2(bp))
    return out[:, 0]


# speedup vs baseline: 3.8386x; 1.1032x over previous
import jax
import jax.numpy as jnp
from jax import lax
from jax.experimental import pallas as pl
from jax.experimental.pallas import tpu as pltpu
from jax.experimental.pallas import tpu_sc as plsc

N = 10000
E = 160000
G = 16
NF = 128

NCORES = 2
NSUB = 16
NW = NCORES * NSUB
SLAB = 624

G_RPW = 2 * E // NW
G_CH = 40
G_NK = G_RPW // G_CH
G_NB = 2

S_RPW = E // NW
S_CH = 40
S_NK = S_RPW // S_CH
S_NB = 5

BE = 4000
NB_E = E // BE
BN = 2000
NB_N = N // BN

_f32 = jnp.float32
_bf16 = jnp.bfloat16


def _relu(v):
    return jnp.maximum(v, 0.0)


def _sc_mesh():
    return plsc.VectorSubcoreMesh(core_axis_name="c", subcore_axis_name="s")


def _by_slab(sid, copy):
    @pl.when(sid < NSUB - 1)
    def _a():
        copy(sid * SLAB, SLAB)

    @pl.when(sid == NSUB - 1)
    def _b():
        copy((NSUB - 1) * SLAB, N - (NSUB - 1) * SLAB)


def _sc_gather(table, idx3):
    def body(table_ref, idx_ref, out_ref, idxv, tshared, *bufs):
        rows = bufs[:G_NB]
        gsems = bufs[G_NB:2 * G_NB]
        wsems = bufs[2 * G_NB:]
        cid = lax.axis_index("c")
        sid = lax.axis_index("s")
        wid = sid * NCORES + cid
        base = wid * G_RPW
        _by_slab(sid, lambda off, sz: pltpu.sync_copy(
            table_ref.at[pl.ds(off, sz)], tshared.at[pl.ds(off, sz)]))
        pltpu.sync_copy(idx_ref.at[wid], idxv)
        plsc.subcore_barrier()
        for b in range(G_NB):
            pltpu.async_copy(tshared.at[idxv.at[b]], rows[b], gsems[b])

        @pl.loop(0, G_NK, step=G_NB)
        def _step(k):
            for b in range(G_NB):
                pltpu.make_async_copy(
                    tshared.at[idxv.at[k + b]], rows[b], gsems[b]).wait()
                dst = out_ref.at[pl.ds(base + (k + b) * G_CH, G_CH)]
                pltpu.async_copy(rows[b], dst, wsems[b])

                @pl.when(k + b + G_NB < G_NK)
                def _nxt():
                    pltpu.make_async_copy(rows[b], dst, wsems[b]).wait()
                    pltpu.async_copy(
                        tshared.at[idxv.at[k + b + G_NB]], rows[b], gsems[b])

        for b in range(G_NB):
            pltpu.make_async_copy(
                rows[b],
                out_ref.at[pl.ds(base + (G_NK - G_NB + b) * G_CH, G_CH)],
                wsems[b]).wait()

    return pl.kernel(
        body,
        out_type=jax.ShapeDtypeStruct((2 * E, 128), _f32),
        mesh=_sc_mesh(),
        scratch_types=[pltpu.VMEM((G_NK, G_CH), jnp.int32),
                       pltpu.VMEM_SHARED((N, 128), _f32)]
        + [pltpu.VMEM((G_CH, 128), _f32) for _ in range(G_NB)]
        + [pltpu.SemaphoreType.DMA for _ in range(2 * G_NB)],
        name="sc_gather_x",
    )(table, idx3)


def _sc_scatter(m, col3, zeros):
    def body(m_ref, col_ref, z_ref, out_ref, idxv, *bufs):
        rows = bufs[:S_NB]
        sems = bufs[S_NB:-1]
        acc = bufs[-1]
        cid = lax.axis_index("c")
        sid = lax.axis_index("s")
        wid = sid * NCORES + cid
        base = wid * S_RPW
        _by_slab(sid, lambda off, sz: pltpu.sync_copy(
            z_ref.at[pl.ds(off, sz)], acc.at[pl.ds(off, sz)]))
        plsc.subcore_barrier()
        pltpu.sync_copy(col_ref.at[wid], idxv)
        for b in range(S_NB):
            pltpu.async_copy(m_ref.at[pl.ds(base + b * S_CH, S_CH)], rows[b], sems[b])

        @pl.loop(0, S_NK, step=S_NB)
        def _step(k):
            for b in range(S_NB):
                pltpu.make_async_copy(
                    m_ref.at[pl.ds(base + (k + b) * S_CH, S_CH)], rows[b], sems[b]).wait()
                pltpu.sync_copy(rows[b], acc.at[idxv.at[k + b]], add=True)

                @pl.when(k + b + S_NB < S_NK)
                def _nxt():
                    pltpu.async_copy(
                        m_ref.at[pl.ds(base + (k + b + S_NB) * S_CH, S_CH)],
                        rows[b], sems[b])

        plsc.subcore_barrier()
        _by_slab(sid, lambda off, sz: pltpu.sync_copy(
            acc.at[pl.ds(off, sz)], out_ref.at[cid, pl.ds(off, sz)]))

    return pl.kernel(
        body,
        out_type=jax.ShapeDtypeStruct((2, N, 128), _f32),
        mesh=_sc_mesh(),
        scratch_types=[pltpu.VMEM((S_NK, S_CH), jnp.int32)]
        + [pltpu.VMEM((S_CH, 128), _f32) for _ in range(S_NB)]
        + [pltpu.SemaphoreType.DMA for _ in range(S_NB)]
        + [pltpu.VMEM_SHARED((N, 128), _f32)],
        name="sc_scatter_m",
    )(m, col3, zeros)


def _sc_counts(col3, zeros, ones128):
    def body(col_ref, z_ref, o_ref, out_ref, idxv, onesv, acc):
        cid = lax.axis_index("c")
        sid = lax.axis_index("s")
        wid = sid * NCORES + cid
        _by_slab(sid, lambda off, sz: pltpu.sync_copy(
            z_ref.at[pl.ds(off, sz)], acc.at[pl.ds(off, sz)]))
        plsc.subcore_barrier()
        pltpu.sync_copy(col_ref.at[wid], idxv)
        pltpu.sync_copy(o_ref, onesv)

        @pl.loop(0, S_NK)
        def _step(k):
            pltpu.sync_copy(onesv, acc.at[idxv.at[k]], add=True)

        plsc.subcore_barrier()
        _by_slab(sid, lambda off, sz: pltpu.sync_copy(
            acc.at[pl.ds(off, sz)], out_ref.at[cid, pl.ds(off, sz)]))

    return pl.kernel(
        body,
        out_type=jax.ShapeDtypeStruct((2, N, 128), _f32),
        mesh=_sc_mesh(),
        scratch_types=[
            pltpu.VMEM((S_NK, S_CH), jnp.int32),
            pltpu.VMEM((S_CH, 128), _f32),
            pltpu.VMEM_SHARED((N, 128), _f32),
        ],
        name="sc_counts",
    )(col3, zeros, ones128)


def _tc_edge(gall, e_in, Ws, Wd, Wea, be, W1x, W1e, b1):
    EF = e_in.shape[1]

    def body(gs_ref, gd_ref, e_ref, Ws_ref, Wd_ref, Wea_ref, be_ref,
             W1x_ref, W1e_ref, b1_ref, eo_ref, m_ref):
        gs = gs_ref[...].astype(_bf16)
        gd = gd_ref[...].astype(_bf16)
        t = jnp.dot(gs, Ws_ref[...], preferred_element_type=_f32)
        t = t + jnp.dot(gd, Wd_ref[...], preferred_element_type=_f32)
        t = t + jnp.dot(e_ref[...], Wea_ref[...], preferred_element_type=_f32)
        e2 = _relu(t + be_ref[...])
        e2b = e2.astype(_bf16)
        mm = jnp.dot(gs, W1x_ref[...], preferred_element_type=_f32)
        mm = mm + jnp.dot(e2b, W1e_ref[...], preferred_element_type=_f32)
        eo_ref[...] = e2b
        m_ref[...] = _relu(mm + b1_ref[...])

    full = lambda a: pl.BlockSpec(a.shape, lambda i: (0,) * a.ndim)
    return pl.pallas_call(
        body,
        grid=(NB_E,),
        in_specs=[
            pl.BlockSpec((BE, 128), lambda i: (i, 0)),
            pl.BlockSpec((BE, 128), lambda i: (i + NB_E, 0)),
            pl.BlockSpec((BE, EF), lambda i: (i, 0)),
            full(Ws), full(Wd), full(Wea), full(be), full(W1x), full(W1e), full(b1),
        ],
        out_specs=[
            pl.BlockSpec((BE, 256), lambda i: (i, 0)),
            pl.BlockSpec((BE, 128), lambda i: (i, 0)),
        ],
        out_shape=[
            jax.ShapeDtypeStruct((E, 256), _bf16),
            jax.ShapeDtypeStruct((E, 128), _f32),
        ],
        name="tc_edge",
    )(gall, gall, e_in, Ws, Wd, Wea, be, W1x, W1e, b1)


def _tc_node(x, parts, cnt_parts, batch3, u, W2x, W2a, b2, Wgu, Wgn, bg):
    has_u = u is not None

    def body(*refs):
        if has_u:
            (x_ref, p_ref, c_ref, b_ref, u_ref, W2x_ref, W2a_ref, b2_ref,
             Wgu_ref, Wgn_ref, bg_ref, xo_ref, uo_ref, nagg_acc, ncnt_acc) = refs
        else:
            (x_ref, p_ref, c_ref, b_ref, W2x_ref, W2a_ref, b2_ref,
             Wgn_ref, bg_ref, xo_ref, uo_ref, nagg_acc, ncnt_acc) = refs
        i = pl.program_id(0)
        cnt = c_ref[0, :, 0:1] + c_ref[1, :, 0:1]
        agg = (p_ref[0] + p_ref[1]) / jnp.maximum(cnt, 1.0)
        xn = jnp.dot(x_ref[...], W2x_ref[...], preferred_element_type=_f32)
        xn = xn + jnp.dot(agg, W2a_ref[...], preferred_element_type=_f32)
        xn = _relu(xn + b2_ref[...])
        xo_ref[...] = xn

        bb = b_ref[0, 0, :]
        oh = (bb[:, None] == lax.broadcasted_iota(jnp.int32, (BN, G), 1))
        ohf = oh.astype(_f32)

        @pl.when(i == 0)
        def _init():
            nagg_acc[...] = jnp.zeros((G, 128), _f32)
            ncnt_acc[...] = jnp.zeros((G, 128), _f32)

        dn = (((0,), (0,)), ((), ()))
        nagg_acc[...] += lax.dot_general(ohf, xn, dn, preferred_element_type=_f32)
        ncnt_acc[...] += lax.dot_general(ohf, jnp.ones((BN, 128), _f32), dn,
                                         preferred_element_type=_f32)

        @pl.when(i == NB_N - 1)
        def _glob():
            nagg = nagg_acc[...] / jnp.maximum(ncnt_acc[...], 1.0)
            gg = jnp.dot(nagg, Wgn_ref[...], preferred_element_type=_f32)
            if has_u:
                gg = gg + jnp.dot(u_ref[...], Wgu_ref[...],
                                  preferred_element_type=_f32)
            uo_ref[...] = _relu(gg + bg_ref[...])

    full = lambda a: pl.BlockSpec(a.shape, lambda i: (0,) * a.ndim)
    ins = [x, parts, cnt_parts, batch3]
    in_specs = [
        pl.BlockSpec((BN, 128), lambda i: (i, 0)),
        pl.BlockSpec((2, BN, 128), lambda i: (0, i, 0)),
        pl.BlockSpec((2, BN, 128), lambda i: (0, i, 0)),
        pl.BlockSpec((1, 1, BN), lambda i: (i, 0, 0)),
    ]
    if has_u:
        ins += [u, W2x, W2a, b2, Wgu, Wgn, bg]
        in_specs += [full(u), full(W2x), full(W2a), full(b2), full(Wgu),
                     full(Wgn), full(bg)]
    else:
        ins += [W2x, W2a, b2, Wgn, bg]
        in_specs += [full(W2x), full(W2a), full(b2), full(Wgn), full(bg)]

    return pl.pallas_call(
        body,
        grid=(NB_N,),
        in_specs=in_specs,
        out_specs=[
            pl.BlockSpec((BN, 128), lambda i: (i, 0)),
            pl.BlockSpec((G, 128), lambda i: (0, 0)),
        ],
        out_shape=[
            jax.ShapeDtypeStruct((N, 128), _f32),
            jax.ShapeDtypeStruct((G, 128), _f32),
        ],
        scratch_shapes=[pltpu.VMEM((G, 128), _f32), pltpu.VMEM((G, 128), _f32)],
        name="tc_node",
    )(*ins)


def _tc_head(ur, up, Wr, br, Wp, bp):
    def body(ur_ref, up_ref, Wr_ref, br_ref, Wp_ref, bp_ref, o_ref):
        rg = jnp.dot(ur_ref[...], Wr_ref[...], preferred_element_type=_f32) + br_ref[...]
        pg = jnp.dot(up_ref[...], Wp_ref[...], preferred_element_type=_f32) + bp_ref[...]
        num = jnp.sum(rg * pg, axis=1, keepdims=True)
        nr = jnp.maximum(jnp.sqrt(jnp.sum(rg * rg, axis=1, keepdims=True)), 1e-8)
        npp = jnp.maximum(jnp.sqrt(jnp.sum(pg * pg, axis=1, keepdims=True)), 1e-8)
        o_ref[...] = jax.nn.sigmoid(num / (nr * npp)) * jnp.ones((1, 128), _f32)

    return pl.pallas_call(
        body,
        out_shape=jax.ShapeDtypeStruct((G, 128), _f32),
        name="tc_head",
    )(ur, up, Wr, br, Wp, bp)


def _row2(v):
    return v.reshape(1, -1)


class _Chain:
    def __init__(self, x, e, u, edge_index, batch, layer_params, zeros, ones128):
        self.x = x
        self.e = e.astype(_bf16)
        self.u = u
        self.idx3 = edge_index.reshape(NW, G_NK, G_CH)
        self.col3 = edge_index[1].reshape(NW, S_NK, S_CH)
        self.batch3 = batch.reshape(NB_N, 1, BN)
        self.zeros = zeros
        self.layers = layer_params
        self.cnt_parts = _sc_counts(self.col3, zeros, ones128)
        self.g = None
        self.m = None
        self.parts = None

    def gather(self):
        self.g = _sc_gather(self.x, self.idx3)

    def edge(self, k):
        We, be = self.layers[k]['edge']
        W1, b1 = self.layers[k]['node1']
        self.e, self.m = _tc_edge(
            self.g, self.e,
            We[:128].astype(_bf16), We[128:256].astype(_bf16),
            We[256:].astype(_bf16), _row2(be),
            W1[:128].astype(_bf16), W1[128:].astype(_bf16), _row2(b1))

    def scatter(self):
        self.parts = _sc_scatter(self.m, self.col3, self.zeros)

    def node(self, k):
        W2, b2 = self.layers[k]['node2']
        Wg, bg = self.layers[k]['glob']
        if self.u is None:
            Wgu, Wgn = None, Wg
        else:
            gu = self.u.shape[1]
            Wgu, Wgn = Wg[:gu], Wg[gu:]
        self.x, self.u = _tc_node(self.x, self.parts, self.cnt_parts,
                                  self.batch3, self.u,
                                  W2[:128], W2[128:], _row2(b2),
                                  Wgu, Wgn, _row2(bg))


def kernel(rx, r_edge_attr, r_edge_index, r_batch, r_g, px, p_edge_attr,
           p_edge_index, p_batch, params):
    zeros = jnp.zeros((N, 128), _f32)
    ones128 = jnp.ones((S_CH, 128), _f32)
    cr = _Chain(rx, r_edge_attr, r_g, r_edge_index, r_batch,
                [params[k] for k in ('r1', 'r2', 'r3', 'r4')], zeros, ones128)
    cp = _Chain(px, p_edge_attr, None, p_edge_index, p_batch,
                [params[k] for k in ('p1', 'p2', 'p3', 'p4')], zeros, ones128)
    for k in range(4):
        cr.gather()
        cp.gather()
        cr.edge(k)
        cr.scatter()
        cp.edge(k)
        cp.scatter()
        cr.node(k)
        cp.node(k)
    Wr, br = params['r_lin1']
    Wp, bp = params['p_lin1']
    out = _tc_head(cr.u, cp.u, Wr, _row2(br), Wp, _row2(bp))
    return out[:, 0]
